# Initial kernel scaffold; baseline (speedup 1.0000x reference)
import jax
import jax.numpy as jnp
from jax.experimental import pallas as pl


def kernel(adj, feat, a1w, a1b, a2w, a2b, lw, lb, fw, fb, aw, ab, pw, pb):
    raise NotImplementedError("write your pallas kernel here")


# speedup vs baseline: 3.9739x; 3.9739x over previous
import math
from functools import partial

import jax
import jax.numpy as jnp
from jax.experimental import pallas as pl
from jax.experimental.pallas import tpu as pltpu

_N = 8
_D = 32
_LAYERS = 2
_BN_EPS = 1e-5
_LN_EPS = 1e-3
_GB = 32
_M = _GB * _N
_NG = 4

_CF0, _CF1, _CAT = 2, 18, 18


def _gcn_kernel(feat_ref, adj_ref, w_ref, p_ref, tile_ref, mask_ref, out_ref):
    f32 = jnp.float32
    misc = p_ref[_LAYERS + 1:_LAYERS + 2, :]
    a1b = misc[:, 0:1]
    a2b = misc[:, 1:2]
    pb = misc[:, 2:3]
    hb = p_ref[_LAYERS:_LAYERS + 1, :]
    pw_row = p_ref[_LAYERS + 2:_LAYERS + 3, 0:16]
    w_fused = w_ref[_LAYERS]

    for u in range(_NG):
        feat = feat_ref[u * _M:(u + 1) * _M, :]
        adj_u = adj_ref[u * _GB:(u + 1) * _GB]
        adj_rows = adj_u.reshape(_M, _N)

        atts = jnp.dot(feat, w_fused, preferred_element_type=f32)
        na = atts[:, 0:1] + a1b
        ea = atts[:, 1:2] + a2b

        nag = na.reshape(_GB, _N)
        e = jnp.exp(nag - jnp.max(nag, axis=1, keepdims=True))
        natt = e / jnp.sum(e, axis=1, keepdims=True)

        adj_sc = ea * adj_rows
        tiled = jnp.dot(adj_sc, tile_ref[...], preferred_element_type=f32)
        bd = tiled * mask_ref[...]

        hidden = na * feat
        for l in range(_LAYERS):
            support = jnp.dot(hidden, w_ref[l], preferred_element_type=f32)
            lb = p_ref[l:l + 1, :]
            out = jnp.dot(bd, support, preferred_element_type=f32) + lb
            hidden = jnp.tanh(out) + hidden

        heads = jnp.dot(hidden, w_fused, preferred_element_type=f32)
        head_out = jnp.dot(bd, heads, preferred_element_type=f32) + hb
        agg = head_out[:, _CF0:_CF1]
        att = jnp.tanh(head_out[:, _CAT:_CAT + 1])

        pf = jnp.sum((att * agg).reshape(_GB, _N, 16), axis=1)
        mu = jnp.mean(pf, axis=-1, keepdims=True)
        var = jnp.mean((pf - mu) ** 2, axis=-1, keepdims=True)
        pf = jnp.tanh((pf - mu) * jax.lax.rsqrt(var + _LN_EPS))

        adj_att = jnp.dot(bd, att, preferred_element_type=f32)
        pm = jnp.tanh(jnp.sum((att * adj_att).reshape(_GB, _N, 1), axis=1))

        sp = jnp.sum(pf * pw_row, axis=-1, keepdims=True)
        pred = pm * sp + pb

        out_ref[u * _GB:(u + 1) * _GB, 0:1] = pred
        out_ref[u * _GB:(u + 1) * _GB, 1:1 + _N] = natt


def _pack(a1w, a1b, a2w, a2b, lw, lb, fw, fb, aw, ab, pw, pb):
    d = _D
    scale = 1.0 / math.sqrt(1.0 + _BN_EPS)
    wf = jnp.zeros((d, d), jnp.float32)
    wf = wf.at[:, 0].set(a1w[:, 0])
    wf = wf.at[:, 1].set(a2w[:, 0])
    wf = wf.at[:, _CF0:_CF1].set(fw * scale)
    wf = wf.at[:, _CAT].set(aw[:, 0] * scale)
    w_slab = jnp.concatenate([lw, wf[None]], axis=0)

    p_slab = jnp.zeros((_LAYERS + 3, d), jnp.float32)
    p_slab = p_slab.at[:_LAYERS, :].set(lb[:, 0, :])
    p_slab = p_slab.at[_LAYERS, _CF0:_CF1].set(fb[0] * scale)
    p_slab = p_slab.at[_LAYERS, _CAT].set(ab[0, 0] * scale)
    p_slab = p_slab.at[_LAYERS + 1, 0].set(a1b[0, 0])
    p_slab = p_slab.at[_LAYERS + 1, 1].set(a2b[0, 0])
    p_slab = p_slab.at[_LAYERS + 1, 2].set(pb[0, 0])
    p_slab = p_slab.at[_LAYERS + 2, :16].set(pw[:, 0])
    return w_slab, p_slab


def kernel(adj, feat, a1w, a1b, a2w, a2b, lw, lb, fw, fb, aw, ab, pw, pb):
    b = adj.shape[0]
    w_slab, p_slab = _pack(a1w, a1b, a2w, a2b, lw, lb, fw, fb, aw, ab, pw, pb)
    feat2 = feat.reshape(b * _N, _D)

    tile_m = jnp.tile(jnp.eye(_N, dtype=jnp.float32), (1, _GB))
    mask = jnp.kron(jnp.eye(_GB, dtype=jnp.float32),
                    jnp.ones((_N, _N), jnp.float32))

    nsteps = b // (_NG * _GB)
    out = pl.pallas_call(
        _gcn_kernel,
        out_shape=jax.ShapeDtypeStruct((b, 1 + _N), jnp.float32),
        grid=(nsteps,),
        in_specs=[
            pl.BlockSpec((_NG * _M, _D), lambda i: (i, 0)),
            pl.BlockSpec((_NG * _GB, _N, _N), lambda i: (i, 0, 0)),
            pl.BlockSpec((_LAYERS + 1, _D, _D), lambda i: (0, 0, 0)),
            pl.BlockSpec((_LAYERS + 3, _D), lambda i: (0, 0)),
            pl.BlockSpec((_N, _M), lambda i: (0, 0)),
            pl.BlockSpec((_M, _M), lambda i: (0, 0)),
        ],
        out_specs=pl.BlockSpec((_NG * _GB, 1 + _N), lambda i: (i, 0)),
        compiler_params=pltpu.CompilerParams(
            dimension_semantics=("parallel",),
        ),
    )(feat2, adj, w_slab, p_slab, tile_m, mask)

    return out[:, 0:1], out[:, 1:]


# speedup vs baseline: 4.9750x; 1.2519x over previous
import math

import jax
import jax.numpy as jnp
from jax.experimental import pallas as pl
from jax.experimental.pallas import tpu as pltpu

_N = 8
_D = 32
_LAYERS = 2
_BN_EPS = 1e-5
_LN_EPS = 1e-3
_GB = 32
_M = _GB * _N
_NG = 4

_CF0, _CF1, _CAT = 2, 18, 18


def _g8(x, lanemod, op, k):
    a = pltpu.roll(x, _M - k, axis=1)
    b = pltpu.roll(x, _N - k, axis=1)
    return op(x, jnp.where(lanemod < _N - k, a, b))


def _gcn_kernel(featT_ref, adjT_ref, w_ref, p_ref, mask_ref, pool_ref,
                natt_ref, pred_ref):
    f32 = jnp.float32
    w0T = w_ref[0]
    w1T = w_ref[1]
    wfT = w_ref[_LAYERS]
    lb0T = p_ref[:, 0:1]
    lb1T = p_ref[:, 1:2]
    hbT = p_ref[:, 2:3]
    a1b = p_ref[0:1, 3:4]
    a2b = p_ref[1:2, 3:4]
    pb = p_ref[2:3, 3:4]
    pw16 = p_ref[0:16, 4:5].reshape(1, 16)

    lanemod = jax.lax.broadcasted_iota(jnp.int32, (1, _M), 1) % _N

    for u in range(_NG):
        featT = featT_ref[:, u * _M:(u + 1) * _M]
        adjT = adjT_ref[:, u * _M:(u + 1) * _M]

        attsT = jnp.dot(wfT, featT, preferred_element_type=f32)
        na = attsT[0:1, :] + a1b
        ea = attsT[1:2, :] + a2b

        mx = na
        for k in (1, 2, 4):
            mx = _g8(mx, lanemod, jnp.maximum, k)
        e = jnp.exp(na - mx)
        s = e
        for k in (1, 2, 4):
            s = _g8(s, lanemod, jnp.add, k)
        natt = e / s
        natt_ref[0, 0:1, u * _M:(u + 1) * _M] = natt

        bdT = pltpu.repeat(ea * adjT, _GB, 0) * mask_ref[...]

        hiddenT = na * featT
        s1 = jnp.dot(w0T, hiddenT, preferred_element_type=f32)
        hiddenT = jnp.tanh(jnp.dot(s1, bdT, preferred_element_type=f32)
                           + lb0T) + hiddenT
        s2 = jnp.dot(w1T, hiddenT, preferred_element_type=f32)
        hiddenT = jnp.tanh(jnp.dot(s2, bdT, preferred_element_type=f32)
                           + lb1T) + hiddenT

        headsT = jnp.dot(wfT, hiddenT, preferred_element_type=f32)
        head_outT = jnp.dot(headsT, bdT, preferred_element_type=f32) + hbT
        aggT = head_outT[_CF0:_CF1, :]
        att = jnp.tanh(head_outT[_CAT:_CAT + 1, :])

        pfT = jnp.dot(att * aggT, pool_ref[...],
                      preferred_element_type=f32)
        mu = jnp.mean(pfT, axis=0, keepdims=True)
        var = jnp.mean((pfT - mu) ** 2, axis=0, keepdims=True)
        pfln = jnp.tanh((pfT - mu) * jax.lax.rsqrt(var + _LN_EPS))

        adj_att = jnp.dot(att, bdT, preferred_element_type=f32)
        pm = jnp.tanh(jnp.dot(att * adj_att, pool_ref[...],
                              preferred_element_type=f32))

        sp = jnp.dot(pw16, pfln, preferred_element_type=f32)
        pred_ref[0, 0:1, u * _GB:(u + 1) * _GB] = pm * sp + pb


def _pack(a1w, a1b, a2w, a2b, lw, lb, fw, fb, aw, ab, pw, pb):
    d = _D
    scale = 1.0 / math.sqrt(1.0 + _BN_EPS)
    wf = jnp.zeros((d, d), jnp.float32)
    wf = wf.at[:, 0].set(a1w[:, 0])
    wf = wf.at[:, 1].set(a2w[:, 0])
    wf = wf.at[:, _CF0:_CF1].set(fw * scale)
    wf = wf.at[:, _CAT].set(aw[:, 0] * scale)
    w_slab = jnp.stack([lw[0].T, lw[1].T, wf.T], axis=0)

    p_slab = jnp.zeros((d, 5), jnp.float32)
    p_slab = p_slab.at[:, 0].set(lb[0, 0, :])
    p_slab = p_slab.at[:, 1].set(lb[1, 0, :])
    hbias = jnp.zeros((d,), jnp.float32)
    hbias = hbias.at[_CF0:_CF1].set(fb[0] * scale)
    hbias = hbias.at[_CAT].set(ab[0, 0] * scale)
    p_slab = p_slab.at[:, 2].set(hbias)
    p_slab = p_slab.at[0, 3].set(a1b[0, 0])
    p_slab = p_slab.at[1, 3].set(a2b[0, 0])
    p_slab = p_slab.at[2, 3].set(pb[0, 0])
    p_slab = p_slab.at[0:16, 4].set(pw[:, 0])
    return w_slab, p_slab


def kernel(adj, feat, a1w, a1b, a2w, a2b, lw, lb, fw, fb, aw, ab, pw, pb):
    b = adj.shape[0]
    w_slab, p_slab = _pack(a1w, a1b, a2w, a2b, lw, lb, fw, fb, aw, ab, pw, pb)

    featT = feat.transpose(2, 0, 1).reshape(_D, b * _N)
    adjT = adj.transpose(2, 0, 1).reshape(_N, b * _N)

    mask = jnp.kron(jnp.eye(_GB, dtype=jnp.float32),
                    jnp.ones((_N, _N), jnp.float32))
    pool = jnp.kron(jnp.eye(_GB, dtype=jnp.float32),
                    jnp.ones((_N, 1), jnp.float32))

    nsteps = b // (_NG * _GB)
    cols = _NG * _M
    natt_out, pred_out = pl.pallas_call(
        _gcn_kernel,
        out_shape=(
            jax.ShapeDtypeStruct((nsteps, 1, cols), jnp.float32),
            jax.ShapeDtypeStruct((nsteps, 1, _NG * _GB), jnp.float32),
        ),
        grid=(nsteps,),
        in_specs=[
            pl.BlockSpec((_D, cols), lambda i: (0, i)),
            pl.BlockSpec((_N, cols), lambda i: (0, i)),
            pl.BlockSpec((_LAYERS + 1, _D, _D), lambda i: (0, 0, 0)),
            pl.BlockSpec((_D, 5), lambda i: (0, 0)),
            pl.BlockSpec((_M, _M), lambda i: (0, 0)),
            pl.BlockSpec((_M, _GB), lambda i: (0, 0)),
        ],
        out_specs=(
            pl.BlockSpec((1, 1, cols), lambda i: (i, 0, 0)),
            pl.BlockSpec((1, 1, _NG * _GB), lambda i: (i, 0, 0)),
        ),
        compiler_params=pltpu.CompilerParams(
            dimension_semantics=("parallel",),
        ),
    )(featT, adjT, w_slab, p_slab, mask, pool)

    return pred_out.reshape(b, 1), natt_out.reshape(b, _N)


# speedup vs baseline: 9.2833x; 1.8660x over previous
import math

import jax
import jax.numpy as jnp
from jax.experimental import pallas as pl
from jax.experimental.pallas import tpu as pltpu

_N = 8
_D = 32
_LAYERS = 2
_BN_EPS = 1e-5
_LN_EPS = 1e-3
_GB = 32
_M = _GB * _N
_NG = 4
_C = _NG * _M

_CF0, _CF1, _CAT = 2, 18, 18


def _g8(x, lanemod, op, k):
    n = x.shape[-1]
    a = pltpu.roll(x, n - k, axis=1)
    b = pltpu.roll(x, _N - k, axis=1)
    return op(x, jnp.where(lanemod < _N - k, a, b))


def _split_dot(a, b_ref):
    f32 = jnp.float32
    a_hi = a.astype(jnp.bfloat16).astype(f32)
    a_lo = a - a_hi
    return (jnp.dot(a_hi, b_ref[...], preferred_element_type=f32)
            + jnp.dot(a_lo, b_ref[...], preferred_element_type=f32))


def _gcn_kernel(featT_ref, adjT_ref, w_ref, p_ref, mask_ref, pool_ref,
                natt_ref, pred_ref):
    f32 = jnp.float32
    w0T = w_ref[0]
    w1T = w_ref[1]
    wfT = w_ref[_LAYERS]
    lb0T = p_ref[:, 0:1]
    lb1T = p_ref[:, 1:2]
    hbT = p_ref[:, 2:3]
    a1b = p_ref[0:1, 3:4]
    a2b = p_ref[1:2, 3:4]
    pb = p_ref[2:3, 3:4]
    pwT = p_ref[0:16, 4:5]

    featT = featT_ref[...]
    adjT = adjT_ref[...]

    attsT = jnp.dot(wfT, featT, preferred_element_type=f32)
    na = attsT[0:1, :] + a1b
    ea = attsT[1:2, :] + a2b

    lanemod = jax.lax.broadcasted_iota(jnp.int32, (1, _C), 1) % _N
    mx = na
    for k in (1, 2, 4):
        mx = _g8(mx, lanemod, jnp.maximum, k)
    e = jnp.exp(na - mx)
    s = e
    for k in (1, 2, 4):
        s = _g8(s, lanemod, jnp.add, k)
    natt_ref[0, 0:1, :] = e / s

    z = ea * adjT
    bdT = [pltpu.repeat(z[:, u * _M:(u + 1) * _M], _GB, 0) * mask_ref[...]
           for u in range(_NG)]

    def bd_apply(x, bias):
        parts = [jnp.dot(x[:, u * _M:(u + 1) * _M], bdT[u],
                         preferred_element_type=f32) for u in range(_NG)]
        return jnp.concatenate(parts, axis=1) + bias

    hid = na * featT
    s1 = jnp.dot(w0T, hid, preferred_element_type=f32)
    hid = jnp.tanh(bd_apply(s1, lb0T)) + hid
    s2 = jnp.dot(w1T, hid, preferred_element_type=f32)
    hid = jnp.tanh(bd_apply(s2, lb1T)) + hid

    heads = jnp.dot(wfT, hid, preferred_element_type=f32)
    ho = bd_apply(heads, hbT)
    agg = ho[_CF0:_CF1, :]
    att = jnp.tanh(ho[_CAT:_CAT + 1, :])

    wagg = att * agg
    pf = jnp.concatenate(
        [_split_dot(wagg[:, u * _M:(u + 1) * _M], pool_ref)
         for u in range(_NG)], axis=1)
    mu = jnp.mean(pf, axis=0, keepdims=True)
    var = jnp.mean((pf - mu) ** 2, axis=0, keepdims=True)
    pfln = jnp.tanh((pf - mu) * jax.lax.rsqrt(var + _LN_EPS))

    aa = jnp.concatenate(
        [jnp.dot(att[:, u * _M:(u + 1) * _M], bdT[u],
                 preferred_element_type=f32) for u in range(_NG)], axis=1)
    q = att * aa
    pm = jnp.tanh(jnp.concatenate(
        [_split_dot(q[:, u * _M:(u + 1) * _M], pool_ref)
         for u in range(_NG)], axis=1))

    sp = jnp.sum(pfln * pwT, axis=0, keepdims=True)
    pred_ref[0, 0:1, :] = pm * sp + pb


def _pack(a1w, a1b, a2w, a2b, lw, lb, fw, fb, aw, ab, pw, pb):
    d = _D
    scale = 1.0 / math.sqrt(1.0 + _BN_EPS)
    wf = jnp.zeros((d, d), jnp.float32)
    wf = wf.at[:, 0].set(a1w[:, 0])
    wf = wf.at[:, 1].set(a2w[:, 0])
    wf = wf.at[:, _CF0:_CF1].set(fw * scale)
    wf = wf.at[:, _CAT].set(aw[:, 0] * scale)
    w_slab = jnp.stack([lw[0].T, lw[1].T, wf.T], axis=0)

    p_slab = jnp.zeros((d, 5), jnp.float32)
    p_slab = p_slab.at[:, 0].set(lb[0, 0, :])
    p_slab = p_slab.at[:, 1].set(lb[1, 0, :])
    hbias = jnp.zeros((d,), jnp.float32)
    hbias = hbias.at[_CF0:_CF1].set(fb[0] * scale)
    hbias = hbias.at[_CAT].set(ab[0, 0] * scale)
    p_slab = p_slab.at[:, 2].set(hbias)
    p_slab = p_slab.at[0, 3].set(a1b[0, 0])
    p_slab = p_slab.at[1, 3].set(a2b[0, 0])
    p_slab = p_slab.at[2, 3].set(pb[0, 0])
    p_slab = p_slab.at[0:16, 4].set(pw[:, 0])
    return w_slab, p_slab


def kernel(adj, feat, a1w, a1b, a2w, a2b, lw, lb, fw, fb, aw, ab, pw, pb):
    b = adj.shape[0]
    w_slab, p_slab = _pack(a1w, a1b, a2w, a2b, lw, lb, fw, fb, aw, ab, pw, pb)

    featT = feat.transpose(2, 0, 1).reshape(_D, b * _N)
    adjT = adj.transpose(2, 0, 1).reshape(_N, b * _N)

    mask = jnp.kron(jnp.eye(_GB, dtype=jnp.float32),
                    jnp.ones((_N, _N), jnp.float32))
    pool = jnp.kron(jnp.eye(_GB, dtype=jnp.float32),
                    jnp.ones((_N, 1), jnp.float32))

    nsteps = b // (_NG * _GB)
    natt_out, pred_out = pl.pallas_call(
        _gcn_kernel,
        out_shape=(
            jax.ShapeDtypeStruct((nsteps, 1, _C), jnp.float32),
            jax.ShapeDtypeStruct((nsteps, 1, _NG * _GB), jnp.float32),
        ),
        grid=(nsteps,),
        in_specs=[
            pl.BlockSpec((_D, _C), lambda i: (0, i)),
            pl.BlockSpec((_N, _C), lambda i: (0, i)),
            pl.BlockSpec((_LAYERS + 1, _D, _D), lambda i: (0, 0, 0)),
            pl.BlockSpec((_D, 5), lambda i: (0, 0)),
            pl.BlockSpec((_M, _M), lambda i: (0, 0)),
            pl.BlockSpec((_M, _GB), lambda i: (0, 0)),
        ],
        out_specs=(
            pl.BlockSpec((1, 1, _C), lambda i: (i, 0, 0)),
            pl.BlockSpec((1, 1, _NG * _GB), lambda i: (i, 0, 0)),
        ),
        compiler_params=pltpu.CompilerParams(
            dimension_semantics=("parallel",),
        ),
    )(featT, adjT, w_slab, p_slab, mask, pool)

    return pred_out.reshape(b, 1), natt_out.reshape(b, _N)


# speedup vs baseline: 11.2899x; 1.2162x over previous
import math

import jax
import jax.numpy as jnp
from jax.experimental import pallas as pl
from jax.experimental.pallas import tpu as pltpu

_N = 8
_D = 32
_LAYERS = 2
_BN_EPS = 1e-5
_LN_EPS = 1e-3
_GB = 32
_M = _GB * _N
_NG = 8
_C = _NG * _M

_CF0, _CF1, _CAT = 2, 18, 18


def _g8(x, lanemod, op, k):
    n = x.shape[-1]
    a = pltpu.roll(x, n - k, axis=1)
    b = pltpu.roll(x, _N - k, axis=1)
    return op(x, jnp.where(lanemod < _N - k, a, b))


def _split_dot(a, b_ref):
    f32 = jnp.float32
    a_hi = a.astype(jnp.bfloat16).astype(f32)
    a_lo = a - a_hi
    return (jnp.dot(a_hi, b_ref[...], preferred_element_type=f32)
            + jnp.dot(a_lo, b_ref[...], preferred_element_type=f32))


def _gcn_kernel(featT_ref, adjT_ref, w_ref, p_ref, mask_ref, pool_ref,
                natt_ref, pred_ref):
    f32 = jnp.float32
    w0T = w_ref[0]
    w1T = w_ref[1]
    wfT = w_ref[_LAYERS]
    lb0T = p_ref[:, 0:1]
    lb1T = p_ref[:, 1:2]
    hbT = p_ref[:, 2:3]
    a1b = p_ref[0:1, 3:4]
    a2b = p_ref[1:2, 3:4]
    pb = p_ref[2:3, 3:4]
    pwT = p_ref[0:16, 4:5]

    featT = featT_ref[...]
    adjT = adjT_ref[...]

    attsT = jnp.dot(wfT, featT, preferred_element_type=f32)
    na = attsT[0:1, :] + a1b
    ea = attsT[1:2, :] + a2b

    lanemod = jax.lax.broadcasted_iota(jnp.int32, (1, _C), 1) % _N
    mx = na
    for k in (1, 2, 4):
        mx = _g8(mx, lanemod, jnp.maximum, k)
    e = jnp.exp(na - mx)
    s = e
    for k in (1, 2, 4):
        s = _g8(s, lanemod, jnp.add, k)
    natt_ref[0, 0:1, :] = e / s

    z = ea * adjT
    bdT = [pltpu.repeat(z[:, u * _M:(u + 1) * _M], _GB, 0) * mask_ref[...]
           for u in range(_NG)]

    def bd_apply(x, bias):
        parts = [jnp.dot(x[:, u * _M:(u + 1) * _M], bdT[u],
                         preferred_element_type=f32) for u in range(_NG)]
        return jnp.concatenate(parts, axis=1) + bias

    hid = na * featT
    s1 = jnp.dot(w0T, hid, preferred_element_type=f32)
    hid = jnp.tanh(bd_apply(s1, lb0T)) + hid
    s2 = jnp.dot(w1T, hid, preferred_element_type=f32)
    hid = jnp.tanh(bd_apply(s2, lb1T)) + hid

    heads = jnp.dot(wfT, hid, preferred_element_type=f32)
    ho = bd_apply(heads, hbT)
    agg = ho[_CF0:_CF1, :]
    att = jnp.tanh(ho[_CAT:_CAT + 1, :])

    wagg = att * agg
    pf = jnp.concatenate(
        [_split_dot(wagg[:, u * _M:(u + 1) * _M], pool_ref)
         for u in range(_NG)], axis=1)
    mu = jnp.mean(pf, axis=0, keepdims=True)
    var = jnp.mean((pf - mu) ** 2, axis=0, keepdims=True)
    pfln = jnp.tanh((pf - mu) * jax.lax.rsqrt(var + _LN_EPS))

    aa = jnp.concatenate(
        [jnp.dot(att[:, u * _M:(u + 1) * _M], bdT[u],
                 preferred_element_type=f32) for u in range(_NG)], axis=1)
    q = att * aa
    pm = jnp.tanh(jnp.concatenate(
        [_split_dot(q[:, u * _M:(u + 1) * _M], pool_ref)
         for u in range(_NG)], axis=1))

    sp = jnp.sum(pfln * pwT, axis=0, keepdims=True)
    pred_ref[0, 0:1, :] = pm * sp + pb


def _pack(a1w, a1b, a2w, a2b, lw, lb, fw, fb, aw, ab, pw, pb):
    d = _D
    scale = 1.0 / math.sqrt(1.0 + _BN_EPS)
    wf = jnp.zeros((d, d), jnp.float32)
    wf = wf.at[:, 0].set(a1w[:, 0])
    wf = wf.at[:, 1].set(a2w[:, 0])
    wf = wf.at[:, _CF0:_CF1].set(fw * scale)
    wf = wf.at[:, _CAT].set(aw[:, 0] * scale)
    w_slab = jnp.stack([lw[0].T, lw[1].T, wf.T], axis=0)

    p_slab = jnp.zeros((d, 5), jnp.float32)
    p_slab = p_slab.at[:, 0].set(lb[0, 0, :])
    p_slab = p_slab.at[:, 1].set(lb[1, 0, :])
    hbias = jnp.zeros((d,), jnp.float32)
    hbias = hbias.at[_CF0:_CF1].set(fb[0] * scale)
    hbias = hbias.at[_CAT].set(ab[0, 0] * scale)
    p_slab = p_slab.at[:, 2].set(hbias)
    p_slab = p_slab.at[0, 3].set(a1b[0, 0])
    p_slab = p_slab.at[1, 3].set(a2b[0, 0])
    p_slab = p_slab.at[2, 3].set(pb[0, 0])
    p_slab = p_slab.at[0:16, 4].set(pw[:, 0])
    return w_slab, p_slab


def kernel(adj, feat, a1w, a1b, a2w, a2b, lw, lb, fw, fb, aw, ab, pw, pb):
    b = adj.shape[0]
    w_slab, p_slab = _pack(a1w, a1b, a2w, a2b, lw, lb, fw, fb, aw, ab, pw, pb)

    featT = feat.transpose(2, 0, 1).reshape(_D, b * _N)
    adjT = adj.transpose(2, 0, 1).reshape(_N, b * _N)

    mask = jnp.kron(jnp.eye(_GB, dtype=jnp.float32),
                    jnp.ones((_N, _N), jnp.float32))
    pool = jnp.kron(jnp.eye(_GB, dtype=jnp.float32),
                    jnp.ones((_N, 1), jnp.float32))

    nsteps = b // (_NG * _GB)
    natt_out, pred_out = pl.pallas_call(
        _gcn_kernel,
        out_shape=(
            jax.ShapeDtypeStruct((nsteps, 1, _C), jnp.float32),
            jax.ShapeDtypeStruct((nsteps, 1, _NG * _GB), jnp.float32),
        ),
        grid=(nsteps,),
        in_specs=[
            pl.BlockSpec((_D, _C), lambda i: (0, i)),
            pl.BlockSpec((_N, _C), lambda i: (0, i)),
            pl.BlockSpec((_LAYERS + 1, _D, _D), lambda i: (0, 0, 0)),
            pl.BlockSpec((_D, 5), lambda i: (0, 0)),
            pl.BlockSpec((_M, _M), lambda i: (0, 0)),
            pl.BlockSpec((_M, _GB), lambda i: (0, 0)),
        ],
        out_specs=(
            pl.BlockSpec((1, 1, _C), lambda i: (i, 0, 0)),
            pl.BlockSpec((1, 1, _NG * _GB), lambda i: (i, 0, 0)),
        ),
        compiler_params=pltpu.CompilerParams(
            dimension_semantics=("parallel",),
        ),
    )(featT, adjT, w_slab, p_slab, mask, pool)

    return pred_out.reshape(b, 1), natt_out.reshape(b, _N)


# speedup vs baseline: 12.1103x; 1.0727x over previous
import math

import jax
import jax.numpy as jnp
from jax.experimental import pallas as pl
from jax.experimental.pallas import tpu as pltpu

_N = 8
_D = 32
_LAYERS = 2
_BN_EPS = 1e-5
_LN_EPS = 1e-3
_GB = 32
_M = _GB * _N
_NG = 16
_C = _NG * _M

_CF0, _CF1, _CAT = 2, 18, 18


def _g8(x, lanemod, op, k):
    n = x.shape[-1]
    a = pltpu.roll(x, n - k, axis=1)
    b = pltpu.roll(x, _N - k, axis=1)
    return op(x, jnp.where(lanemod < _N - k, a, b))


def _split_dot(a, b_ref):
    f32 = jnp.float32
    a_hi = a.astype(jnp.bfloat16).astype(f32)
    a_lo = a - a_hi
    return (jnp.dot(a_hi, b_ref[...], preferred_element_type=f32)
            + jnp.dot(a_lo, b_ref[...], preferred_element_type=f32))


def _gcn_kernel(featT_ref, adjT_ref, w_ref, p_ref, mask_ref, pool_ref,
                natt_ref, pred_ref):
    f32 = jnp.float32
    w0T = w_ref[0]
    w1T = w_ref[1]
    wfT = w_ref[_LAYERS]
    lb0T = p_ref[:, 0:1]
    lb1T = p_ref[:, 1:2]
    hbT = p_ref[:, 2:3]
    a1b = p_ref[0:1, 3:4]
    a2b = p_ref[1:2, 3:4]
    pb = p_ref[2:3, 3:4]
    pwT = p_ref[0:16, 4:5]

    featT = featT_ref[...]
    adjT = adjT_ref[...]

    attsT = jnp.dot(wfT, featT, preferred_element_type=f32)
    na = attsT[0:1, :] + a1b
    ea = attsT[1:2, :] + a2b

    lanemod = jax.lax.broadcasted_iota(jnp.int32, (1, _C), 1) % _N
    mx = na
    for k in (1, 2, 4):
        mx = _g8(mx, lanemod, jnp.maximum, k)
    e = jnp.exp(na - mx)
    s = e
    for k in (1, 2, 4):
        s = _g8(s, lanemod, jnp.add, k)
    natt_ref[0, 0:1, :] = e / s

    z = ea * adjT
    bdT = [pltpu.repeat(z[:, u * _M:(u + 1) * _M], _GB, 0) * mask_ref[...]
           for u in range(_NG)]

    def bd_apply(x, bias):
        parts = [jnp.dot(x[:, u * _M:(u + 1) * _M], bdT[u],
                         preferred_element_type=f32) for u in range(_NG)]
        return jnp.concatenate(parts, axis=1) + bias

    hid = na * featT
    s1 = jnp.dot(w0T, hid, preferred_element_type=f32)
    hid = jnp.tanh(bd_apply(s1, lb0T)) + hid
    s2 = jnp.dot(w1T, hid, preferred_element_type=f32)
    hid = jnp.tanh(bd_apply(s2, lb1T)) + hid

    heads = jnp.dot(wfT, hid, preferred_element_type=f32)
    ho = bd_apply(heads, hbT)
    agg = ho[_CF0:_CF1, :]
    att = jnp.tanh(ho[_CAT:_CAT + 1, :])

    wagg = att * agg
    pf = jnp.concatenate(
        [_split_dot(wagg[:, u * _M:(u + 1) * _M], pool_ref)
         for u in range(_NG)], axis=1)
    mu = jnp.mean(pf, axis=0, keepdims=True)
    var = jnp.mean((pf - mu) ** 2, axis=0, keepdims=True)
    pfln = jnp.tanh((pf - mu) * jax.lax.rsqrt(var + _LN_EPS))

    aa = jnp.concatenate(
        [jnp.dot(att[:, u * _M:(u + 1) * _M], bdT[u],
                 preferred_element_type=f32) for u in range(_NG)], axis=1)
    q = att * aa
    pm = jnp.tanh(jnp.concatenate(
        [_split_dot(q[:, u * _M:(u + 1) * _M], pool_ref)
         for u in range(_NG)], axis=1))

    sp = jnp.sum(pfln * pwT, axis=0, keepdims=True)
    pred_ref[0, 0:1, :] = pm * sp + pb


def _pack(a1w, a1b, a2w, a2b, lw, lb, fw, fb, aw, ab, pw, pb):
    d = _D
    scale = 1.0 / math.sqrt(1.0 + _BN_EPS)
    wf = jnp.zeros((d, d), jnp.float32)
    wf = wf.at[:, 0].set(a1w[:, 0])
    wf = wf.at[:, 1].set(a2w[:, 0])
    wf = wf.at[:, _CF0:_CF1].set(fw * scale)
    wf = wf.at[:, _CAT].set(aw[:, 0] * scale)
    w_slab = jnp.stack([lw[0].T, lw[1].T, wf.T], axis=0)

    p_slab = jnp.zeros((d, 5), jnp.float32)
    p_slab = p_slab.at[:, 0].set(lb[0, 0, :])
    p_slab = p_slab.at[:, 1].set(lb[1, 0, :])
    hbias = jnp.zeros((d,), jnp.float32)
    hbias = hbias.at[_CF0:_CF1].set(fb[0] * scale)
    hbias = hbias.at[_CAT].set(ab[0, 0] * scale)
    p_slab = p_slab.at[:, 2].set(hbias)
    p_slab = p_slab.at[0, 3].set(a1b[0, 0])
    p_slab = p_slab.at[1, 3].set(a2b[0, 0])
    p_slab = p_slab.at[2, 3].set(pb[0, 0])
    p_slab = p_slab.at[0:16, 4].set(pw[:, 0])
    return w_slab, p_slab


def kernel(adj, feat, a1w, a1b, a2w, a2b, lw, lb, fw, fb, aw, ab, pw, pb):
    b = adj.shape[0]
    w_slab, p_slab = _pack(a1w, a1b, a2w, a2b, lw, lb, fw, fb, aw, ab, pw, pb)

    featT = feat.transpose(2, 0, 1).reshape(_D, b * _N)
    adjT = adj.transpose(2, 0, 1).reshape(_N, b * _N)

    mask = jnp.kron(jnp.eye(_GB, dtype=jnp.float32),
                    jnp.ones((_N, _N), jnp.float32))
    pool = jnp.kron(jnp.eye(_GB, dtype=jnp.float32),
                    jnp.ones((_N, 1), jnp.float32))

    nsteps = b // (_NG * _GB)
    natt_out, pred_out = pl.pallas_call(
        _gcn_kernel,
        out_shape=(
            jax.ShapeDtypeStruct((nsteps, 1, _C), jnp.float32),
            jax.ShapeDtypeStruct((nsteps, 1, _NG * _GB), jnp.float32),
        ),
        grid=(nsteps,),
        in_specs=[
            pl.BlockSpec((_D, _C), lambda i: (0, i)),
            pl.BlockSpec((_N, _C), lambda i: (0, i)),
            pl.BlockSpec((_LAYERS + 1, _D, _D), lambda i: (0, 0, 0)),
            pl.BlockSpec((_D, 5), lambda i: (0, 0)),
            pl.BlockSpec((_M, _M), lambda i: (0, 0)),
            pl.BlockSpec((_M, _GB), lambda i: (0, 0)),
        ],
        out_specs=(
            pl.BlockSpec((1, 1, _C), lambda i: (i, 0, 0)),
            pl.BlockSpec((1, 1, _NG * _GB), lambda i: (i, 0, 0)),
        ),
        compiler_params=pltpu.CompilerParams(
            dimension_semantics=("parallel",),
        ),
    )(featT, adjT, w_slab, p_slab, mask, pool)

    return pred_out.reshape(b, 1), natt_out.reshape(b, _N)


# speedup vs baseline: 16.9412x; 1.3989x over previous
import math

import jax
import jax.numpy as jnp
from jax.experimental import pallas as pl
from jax.experimental.pallas import tpu as pltpu

_N = 8
_D = 32
_LAYERS = 2
_BN_EPS = 1e-5
_LN_EPS = 1e-3
_GB = 32
_M = _GB * _N
_NG = 16
_C = _NG * _M

_CF0, _CF1, _CAT = 2, 18, 18


def _g8(x, lanemod, op, k):
    n = x.shape[-1]
    a = pltpu.roll(x, n - k, axis=1)
    b = pltpu.roll(x, _N - k, axis=1)
    return op(x, jnp.where(lanemod < _N - k, a, b))


def _split_dot(a, b_ref):
    f32 = jnp.float32
    a_hi = a.astype(jnp.bfloat16).astype(f32)
    a_lo = a - a_hi
    return (jnp.dot(a_hi, b_ref[...], preferred_element_type=f32)
            + jnp.dot(a_lo, b_ref[...], preferred_element_type=f32))


def _gcn_kernel(featT_ref, adjT_ref, w_ref, p_ref, mask_ref, pool_ref,
                natt_ref, pred_ref):
    f32 = jnp.float32
    w0T = w_ref[0]
    w1T = w_ref[1]
    wfT = w_ref[_LAYERS]
    lb0T = p_ref[:, 0:1]
    lb1T = p_ref[:, 1:2]
    hbT = p_ref[:, 2:3]
    a1b = p_ref[0:1, 3:4]
    a2b = p_ref[1:2, 3:4]
    pb = p_ref[2:3, 3:4]
    pwT = p_ref[0:16, 4:5]

    featT = jnp.transpose(featT_ref[...], (1, 0))
    adjT = jnp.transpose(adjT_ref[...].reshape(_C, _N), (1, 0))

    attsT = jnp.dot(wfT, featT, preferred_element_type=f32)
    na = attsT[0:1, :] + a1b
    ea = attsT[1:2, :] + a2b

    lanemod = jax.lax.broadcasted_iota(jnp.int32, (1, _C), 1) % _N
    mx = na
    for k in (1, 2, 4):
        mx = _g8(mx, lanemod, jnp.maximum, k)
    e = jnp.exp(na - mx)
    s = e
    for k in (1, 2, 4):
        s = _g8(s, lanemod, jnp.add, k)
    natt_ref[0, 0:1, :] = e / s

    z = ea * adjT
    bdT = [pltpu.repeat(z[:, u * _M:(u + 1) * _M], _GB, 0) * mask_ref[...]
           for u in range(_NG)]

    def bd_apply(x, bias):
        parts = [jnp.dot(x[:, u * _M:(u + 1) * _M], bdT[u],
                         preferred_element_type=f32) for u in range(_NG)]
        return jnp.concatenate(parts, axis=1) + bias

    hid = na * featT
    s1 = jnp.dot(w0T, hid, preferred_element_type=f32)
    hid = jnp.tanh(bd_apply(s1, lb0T)) + hid
    s2 = jnp.dot(w1T, hid, preferred_element_type=f32)
    hid = jnp.tanh(bd_apply(s2, lb1T)) + hid

    heads = jnp.dot(wfT, hid, preferred_element_type=f32)
    ho = bd_apply(heads, hbT)
    agg = ho[_CF0:_CF1, :]
    att = jnp.tanh(ho[_CAT:_CAT + 1, :])

    wagg = att * agg
    pf = jnp.concatenate(
        [_split_dot(wagg[:, u * _M:(u + 1) * _M], pool_ref)
         for u in range(_NG)], axis=1)
    mu = jnp.mean(pf, axis=0, keepdims=True)
    var = jnp.mean((pf - mu) ** 2, axis=0, keepdims=True)
    pfln = jnp.tanh((pf - mu) * jax.lax.rsqrt(var + _LN_EPS))

    aa = jnp.concatenate(
        [jnp.dot(att[:, u * _M:(u + 1) * _M], bdT[u],
                 preferred_element_type=f32) for u in range(_NG)], axis=1)
    q = att * aa
    pm = jnp.tanh(jnp.concatenate(
        [_split_dot(q[:, u * _M:(u + 1) * _M], pool_ref)
         for u in range(_NG)], axis=1))

    sp = jnp.sum(pfln * pwT, axis=0, keepdims=True)
    pred_ref[0, 0:1, :] = pm * sp + pb


def _pack(a1w, a1b, a2w, a2b, lw, lb, fw, fb, aw, ab, pw, pb):
    d = _D
    scale = 1.0 / math.sqrt(1.0 + _BN_EPS)
    wf = jnp.zeros((d, d), jnp.float32)
    wf = wf.at[:, 0].set(a1w[:, 0])
    wf = wf.at[:, 1].set(a2w[:, 0])
    wf = wf.at[:, _CF0:_CF1].set(fw * scale)
    wf = wf.at[:, _CAT].set(aw[:, 0] * scale)
    w_slab = jnp.stack([lw[0].T, lw[1].T, wf.T], axis=0)

    p_slab = jnp.zeros((d, 5), jnp.float32)
    p_slab = p_slab.at[:, 0].set(lb[0, 0, :])
    p_slab = p_slab.at[:, 1].set(lb[1, 0, :])
    hbias = jnp.zeros((d,), jnp.float32)
    hbias = hbias.at[_CF0:_CF1].set(fb[0] * scale)
    hbias = hbias.at[_CAT].set(ab[0, 0] * scale)
    p_slab = p_slab.at[:, 2].set(hbias)
    p_slab = p_slab.at[0, 3].set(a1b[0, 0])
    p_slab = p_slab.at[1, 3].set(a2b[0, 0])
    p_slab = p_slab.at[2, 3].set(pb[0, 0])
    p_slab = p_slab.at[0:16, 4].set(pw[:, 0])
    return w_slab, p_slab


def kernel(adj, feat, a1w, a1b, a2w, a2b, lw, lb, fw, fb, aw, ab, pw, pb):
    b = adj.shape[0]
    w_slab, p_slab = _pack(a1w, a1b, a2w, a2b, lw, lb, fw, fb, aw, ab, pw, pb)

    feat2 = feat.reshape(b * _N, _D)

    mask = jnp.kron(jnp.eye(_GB, dtype=jnp.float32),
                    jnp.ones((_N, _N), jnp.float32))
    pool = jnp.kron(jnp.eye(_GB, dtype=jnp.float32),
                    jnp.ones((_N, 1), jnp.float32))

    nsteps = b // (_NG * _GB)
    natt_out, pred_out = pl.pallas_call(
        _gcn_kernel,
        out_shape=(
            jax.ShapeDtypeStruct((nsteps, 1, _C), jnp.float32),
            jax.ShapeDtypeStruct((nsteps, 1, _NG * _GB), jnp.float32),
        ),
        grid=(nsteps,),
        in_specs=[
            pl.BlockSpec((_C, _D), lambda i: (i, 0)),
            pl.BlockSpec((_C // _N, _N, _N), lambda i: (i, 0, 0)),
            pl.BlockSpec((_LAYERS + 1, _D, _D), lambda i: (0, 0, 0)),
            pl.BlockSpec((_D, 5), lambda i: (0, 0)),
            pl.BlockSpec((_M, _M), lambda i: (0, 0)),
            pl.BlockSpec((_M, _GB), lambda i: (0, 0)),
        ],
        out_specs=(
            pl.BlockSpec((1, 1, _C), lambda i: (i, 0, 0)),
            pl.BlockSpec((1, 1, _NG * _GB), lambda i: (i, 0, 0)),
        ),
        compiler_params=pltpu.CompilerParams(
            dimension_semantics=("parallel",),
        ),
    )(feat2, adj, w_slab, p_slab, mask, pool)

    return pred_out.reshape(b, 1), natt_out.reshape(b, _N)
